# Initial kernel scaffold; baseline (speedup 1.0000x reference)
import jax
import jax.numpy as jnp
from jax.experimental import pallas as pl


def kernel(inputs, codebook):
    raise NotImplementedError("write your pallas kernel here")


# speedup vs baseline: 1.0167x; 1.0167x over previous
import functools

import jax
import jax.numpy as jnp
from jax.experimental import pallas as pl
from jax.experimental.pallas import tpu as pltpu

NUM_EMBEDDINGS = 1024
EMBEDDING_DIM = 64
COMMITMENT_COST = 0.25
ROWS = 1024


def _vq_body(x_ref, e_ref, q_ref, loss_ref, perp_ref, hist_ref, acc_ref,
             *, nsteps, total_elems, total_rows):
    i = pl.program_id(0)
    x = x_ref[...]
    e = e_ref[...]

    mm = jax.lax.dot_general(x, e, (((1,), (1,)), ((), ())),
                             preferred_element_type=jnp.float32)
    zsq = jnp.sum(x * x, axis=1, keepdims=True)
    esq = jnp.sum(e * e, axis=1, keepdims=True).T
    d = (zsq + esq) - 2.0 * mm

    dmin = jnp.min(d, axis=1, keepdims=True)
    codes = jax.lax.broadcasted_iota(jnp.int32, d.shape, 1)
    masked = jnp.where(d == dmin, codes, NUM_EMBEDDINGS)
    idx = jnp.min(masked, axis=1, keepdims=True)

    oh = (codes == idx).astype(jnp.float32)
    q = jax.lax.dot_general(oh, e, (((1,), (0,)), ((), ())),
                            preferred_element_type=jnp.float32)
    q_ref[...] = q

    step_loss = jnp.sum((q - x) ** 2)
    step_hist = jnp.sum(oh, axis=0, keepdims=True)

    @pl.when(i == 0)
    def _init():
        acc_ref[0, 0] = step_loss
        hist_ref[...] = step_hist

    @pl.when(i > 0)
    def _acc():
        acc_ref[0, 0] += step_loss
        hist_ref[...] += step_hist

    @pl.when(i == nsteps - 1)
    def _finish():
        m = acc_ref[0, 0] / total_elems
        loss_ref[...] = jnp.full((1, 1), m + COMMITMENT_COST * m, jnp.float32)
        p = hist_ref[...] * (1.0 / total_rows)
        ent = jnp.sum(p * jnp.log(p + 1e-10), keepdims=True)
        perp_ref[...] = jnp.exp(-ent)


@jax.jit
def kernel(inputs, codebook):
    B, C, H, W = inputs.shape
    n_rows = B * H * W
    nsteps = n_rows // ROWS
    flat = jnp.transpose(inputs, (0, 2, 3, 1)).reshape(n_rows, C)

    body = functools.partial(
        _vq_body, nsteps=nsteps,
        total_elems=float(n_rows * C), total_rows=float(n_rows))

    q_flat, loss, perp = pl.pallas_call(
        body,
        grid=(nsteps,),
        in_specs=[
            pl.BlockSpec((ROWS, C), lambda i: (i, 0)),
            pl.BlockSpec((NUM_EMBEDDINGS, C), lambda i: (0, 0)),
        ],
        out_specs=[
            pl.BlockSpec((ROWS, C), lambda i: (i, 0)),
            pl.BlockSpec((1, 1), lambda i: (0, 0)),
            pl.BlockSpec((1, 1), lambda i: (0, 0)),
        ],
        out_shape=[
            jax.ShapeDtypeStruct((n_rows, C), jnp.float32),
            jax.ShapeDtypeStruct((1, 1), jnp.float32),
            jax.ShapeDtypeStruct((1, 1), jnp.float32),
        ],
        scratch_shapes=[
            pltpu.VMEM((1, NUM_EMBEDDINGS), jnp.float32),
            pltpu.SMEM((1, 1), jnp.float32),
        ],
    )(flat, codebook)

    quantized = jnp.transpose(q_flat.reshape(B, H, W, C), (0, 3, 1, 2))
    quantized_st = inputs + jax.lax.stop_gradient(quantized - inputs)
    return quantized_st, loss[0, 0], perp[0, 0]


# speedup vs baseline: 1.1918x; 1.1723x over previous
import functools

import jax
import jax.numpy as jnp
from jax.experimental import pallas as pl
from jax.experimental.pallas import tpu as pltpu

NUM_EMBEDDINGS = 1024
EMBEDDING_DIM = 64
COMMITMENT_COST = 0.25


def _vq_body(x_ref, e_ref, qst_ref, loss_ref, perp_ref,
             esq_ref, hist_ref, acc_ref, *, nsteps, total_elems, total_rows):
    i = pl.program_id(0)
    x = x_ref[0]
    e = e_ref[...]

    @pl.when(i == 0)
    def _prep():
        esq_ref[...] = jnp.sum(e * e, axis=1, keepdims=True)

    mm = jax.lax.dot_general(e, x, (((1,), (0,)), ((), ())),
                             preferred_element_type=jnp.float32)
    zsq = jnp.sum(x * x, axis=0, keepdims=True)
    d = (zsq + esq_ref[...]) - 2.0 * mm

    dmin = jnp.min(d, axis=0, keepdims=True)
    codes = jax.lax.broadcasted_iota(jnp.int32, d.shape, 0)
    masked = jnp.where(d == dmin, codes, NUM_EMBEDDINGS)
    idx = jnp.min(masked, axis=0, keepdims=True)

    oh = (masked == idx).astype(jnp.float32)
    q = jax.lax.dot_general(e, oh, (((0,), (0,)), ((), ())),
                            preferred_element_type=jnp.float32)
    diff = q - x
    qst_ref[0] = x + diff

    step_loss = jnp.sum(diff * diff)
    step_hist = jnp.sum(oh, axis=1, keepdims=True)

    @pl.when(i == 0)
    def _init():
        acc_ref[0, 0] = step_loss
        hist_ref[...] = step_hist

    @pl.when(i > 0)
    def _acc():
        acc_ref[0, 0] += step_loss
        hist_ref[...] += step_hist

    @pl.when(i == nsteps - 1)
    def _finish():
        m = acc_ref[0, 0] / total_elems
        loss_ref[...] = jnp.full((1, 1), m + COMMITMENT_COST * m, jnp.float32)
        p = hist_ref[...] * (1.0 / total_rows)
        ent = jnp.sum(p * jnp.log(p + 1e-10), keepdims=True)
        perp_ref[...] = jnp.exp(-ent)


@jax.jit
def kernel(inputs, codebook):
    B, C, H, W = inputs.shape
    pix = H * W
    n_rows = B * pix
    x_nat = inputs.reshape(B, C, pix)

    body = functools.partial(
        _vq_body, nsteps=B,
        total_elems=float(n_rows * C), total_rows=float(n_rows))

    qst, loss, perp = pl.pallas_call(
        body,
        grid=(B,),
        in_specs=[
            pl.BlockSpec((1, C, pix), lambda i: (i, 0, 0)),
            pl.BlockSpec((NUM_EMBEDDINGS, C), lambda i: (0, 0)),
        ],
        out_specs=[
            pl.BlockSpec((1, C, pix), lambda i: (i, 0, 0)),
            pl.BlockSpec((1, 1), lambda i: (0, 0)),
            pl.BlockSpec((1, 1), lambda i: (0, 0)),
        ],
        out_shape=[
            jax.ShapeDtypeStruct((B, C, pix), jnp.float32),
            jax.ShapeDtypeStruct((1, 1), jnp.float32),
            jax.ShapeDtypeStruct((1, 1), jnp.float32),
        ],
        scratch_shapes=[
            pltpu.VMEM((NUM_EMBEDDINGS, 1), jnp.float32),
            pltpu.VMEM((NUM_EMBEDDINGS, 1), jnp.float32),
            pltpu.SMEM((1, 1), jnp.float32),
        ],
    )(x_nat, codebook)

    return qst.reshape(B, C, H, W), loss[0, 0], perp[0, 0]
